# Initial kernel scaffold; baseline (speedup 1.0000x reference)
import functools

import jax
import jax.numpy as jnp
from jax import lax
from jax.experimental import pallas as pl
from jax.experimental.pallas import tpu as pltpu
from jax.experimental.pallas import tpu_sc as plsc

N = 10000
F = 128
E = 320000

NC = 2
NS = 16
NW = NC * NS

BE = 128
E_PAD = 327680
EPT = E_PAD // NW
NBATCH = EPT // BE

N_PAD = 10016
INIT_RPT = N_PAD // NS
OUT_RPT = N // NS

HIST = 10240
HPT = HIST // NS

_mesh = plsc.VectorSubcoreMesh(core_axis_name="c", subcore_axis_name="s")


@functools.partial(
    pl.kernel,
    out_type=jax.ShapeDtypeStruct((NC * HIST,), jnp.float32),
    mesh=_mesh,
    scratch_types=[
        pltpu.VMEM((HIST,), jnp.float32),
        pltpu.VMEM((EPT,), jnp.int32),
        pltpu.VMEM_SHARED((HIST,), jnp.float32),
    ],
)
def _deg_kernel(dst_hbm, out_hbm, hist_v, dst_v, hist_sh):
    c = lax.axis_index("c")
    s = lax.axis_index("s")
    wid = c * NS + s

    def zero_body(i, _):
        hist_v[pl.ds(i * 16, 16)] = jnp.zeros((16,), jnp.float32)
        return 0

    lax.fori_loop(0, HIST // 16, zero_body, 0)

    pltpu.sync_copy(hist_v.at[pl.ds(s * HPT, HPT)],
                    hist_sh.at[pl.ds(s * HPT, HPT)])

    pltpu.sync_copy(dst_hbm.at[pl.ds(wid * (E // NW), E // NW)],
                    dst_v.at[pl.ds(0, E // NW)])
    plsc.subcore_barrier()

    ones = jnp.ones((16,), jnp.float32)

    def acc_body(i, _):
        idx = dst_v[pl.ds(i * 16, 16)]
        plsc.addupdate_scatter(hist_v, [idx], ones)
        return 0

    lax.fori_loop(0, (E // NW) // 16, acc_body, 0)

    pltpu.sync_copy(hist_v, hist_sh, add=True)
    plsc.subcore_barrier()

    pltpu.sync_copy(hist_sh.at[pl.ds(s * HPT, HPT)],
                    out_hbm.at[pl.ds(c * HIST + s * HPT, HPT)])


@functools.partial(
    pl.kernel,
    out_type=jax.ShapeDtypeStruct((NC, N, F), jnp.float32),
    mesh=_mesh,
    scratch_types=[
        pltpu.VMEM((BE,), jnp.int32),
        pltpu.VMEM((BE,), jnp.int32),
        pltpu.VMEM((BE, F), jnp.float32),
        pltpu.VMEM_SHARED((N_PAD, F), jnp.float32),
        pltpu.SemaphoreType.DMA,
    ],
)
def _agg_kernel(xs_hbm, src_hbm, dst_hbm, out_hbm, sidx, didx, rows, acc, sem):
    c = lax.axis_index("c")
    s = lax.axis_index("s")
    wid = c * NS + s

    pltpu.sync_copy(xs_hbm.at[pl.ds(s * INIT_RPT, INIT_RPT)],
                    acc.at[pl.ds(s * INIT_RPT, INIT_RPT)])
    plsc.subcore_barrier()

    def body(j, _):
        base = wid * EPT + j * BE
        pltpu.sync_copy(src_hbm.at[pl.ds(base, BE)], sidx)
        pltpu.async_copy(xs_hbm.at[sidx], rows, sem).wait()
        pltpu.sync_copy(dst_hbm.at[pl.ds(base, BE)], didx)
        pltpu.sync_copy(rows, acc.at[didx], add=True)
        return 0

    lax.fori_loop(0, NBATCH, body, 0)
    plsc.subcore_barrier()

    pltpu.sync_copy(acc.at[pl.ds(s * OUT_RPT, OUT_RPT)],
                    out_hbm.at[c, pl.ds(s * OUT_RPT, OUT_RPT)])


ROWS_BLK = 1000
GRID = N // ROWS_BLK

_row_spec = pl.BlockSpec((ROWS_BLK, F), lambda i: (i, 0))
_col_spec = pl.BlockSpec((ROWS_BLK, 1), lambda i: (i, 0))
_full_spec = pl.BlockSpec((F, F), lambda i: (0, 0))
_bias_spec = pl.BlockSpec((1, F), lambda i: (0, 0))


def _mm_body(x_ref, w_ref, o_ref):
    o_ref[...] = jnp.dot(x_ref[...], w_ref[...],
                         preferred_element_type=jnp.float32)


def _tc_mm(x, w):
    return pl.pallas_call(
        _mm_body,
        grid=(GRID,),
        in_specs=[_row_spec, _full_spec],
        out_specs=_row_spec,
        out_shape=jax.ShapeDtypeStruct((N, F), jnp.float32),
    )(x, w)


def _prep_body(y_ref, d0_ref, d1_ref, xs_ref, dinv_ref):
    deg = d0_ref[...] + d1_ref[...] + 1.0
    dinv = lax.rsqrt(deg)
    dinv_ref[...] = dinv
    xs_ref[...] = y_ref[...] * dinv


def _tc_prep(y, d0, d1):
    return pl.pallas_call(
        _prep_body,
        grid=(GRID,),
        in_specs=[_row_spec, _col_spec, _col_spec],
        out_specs=[_row_spec, _col_spec],
        out_shape=[jax.ShapeDtypeStruct((N, F), jnp.float32),
                   jax.ShapeDtypeStruct((N, 1), jnp.float32)],
    )(y, d0, d1)


def _mid_body(p0_ref, p1_ref, xs_ref, dinv_ref, b_ref, w_ref, o_ref):
    agg = p0_ref[...] + p1_ref[...] - xs_ref[...]
    h = jnp.maximum(agg * dinv_ref[...] + b_ref[...], 0.0)
    o_ref[...] = jnp.dot(h, w_ref[...],
                         preferred_element_type=jnp.float32) * dinv_ref[...]


def _tc_mid(p0, p1, xs, dinv, b, w):
    return pl.pallas_call(
        _mid_body,
        grid=(GRID,),
        in_specs=[_row_spec, _row_spec, _row_spec, _col_spec, _bias_spec,
                  _full_spec],
        out_specs=_row_spec,
        out_shape=jax.ShapeDtypeStruct((N, F), jnp.float32),
    )(p0, p1, xs, dinv, b, w)


def _out_body(q0_ref, q1_ref, xs_ref, dinv_ref, b_ref, o_ref):
    agg = q0_ref[...] + q1_ref[...] - xs_ref[...]
    o_ref[...] = agg * dinv_ref[...] + b_ref[...]


def _tc_out(q0, q1, xs, dinv, b):
    return pl.pallas_call(
        _out_body,
        grid=(GRID,),
        in_specs=[_row_spec, _row_spec, _row_spec, _col_spec, _bias_spec],
        out_specs=_row_spec,
        out_shape=jax.ShapeDtypeStruct((N, F), jnp.float32),
    )(q0, q1, xs, dinv, b)


def kernel(x, edge_index, W1, b1, W2, b2):
    src = edge_index[0]
    dst = edge_index[1]

    pad = jnp.full((E_PAD - E,), N, dtype=jnp.int32)
    src_p = jnp.concatenate([src, pad])
    dst_p = jnp.concatenate([dst, pad])

    d_parts = _deg_kernel(dst)
    d0 = d_parts[0:N].reshape(N, 1)
    d1 = d_parts[HIST:HIST + N].reshape(N, 1)

    zpad = jnp.zeros((N_PAD - N, F), jnp.float32)

    y1 = _tc_mm(x, W1)
    xs1, dinv = _tc_prep(y1, d0, d1)
    p = _agg_kernel(jnp.concatenate([xs1, zpad]), src_p, dst_p)
    xs2 = _tc_mid(p[0], p[1], xs1, dinv, b1.reshape(1, F), W2)
    q = _agg_kernel(jnp.concatenate([xs2, zpad]), src_p, dst_p)
    return _tc_out(q[0], q[1], xs2, dinv, b2.reshape(1, F))


# speedup vs baseline: 8.0877x; 8.0877x over previous
import functools

import jax
import jax.numpy as jnp
from jax import lax
from jax.experimental import pallas as pl
from jax.experimental.pallas import tpu as pltpu
from jax.experimental.pallas import tpu_sc as plsc

N = 10000
F = 128
E = 320000

NC = 2
NS = 16
NW = NC * NS

BE = 128
E_PAD = 327680
EPT = E_PAD // NW
NBATCH = EPT // BE

N_PAD = 10240
INIT_RPT = N_PAD // NS
OUT_RPT = N_PAD // NS

HIST = 10240

_mesh = plsc.VectorSubcoreMesh(core_axis_name="c", subcore_axis_name="s")


@functools.partial(
    pl.kernel,
    out_type=jax.ShapeDtypeStruct((NW * HIST,), jnp.float32),
    mesh=_mesh,
    scratch_types=[
        pltpu.VMEM((HIST,), jnp.float32),
        pltpu.VMEM((EPT,), jnp.int32),
    ],
    compiler_params=pltpu.CompilerParams(needs_layout_passes=False),
)
def _deg_kernel(dst_hbm, zeros_hbm, out_hbm, hist_v, dst_v):
    c = lax.axis_index("c")
    s = lax.axis_index("s")
    wid = c * NS + s

    pltpu.sync_copy(zeros_hbm, hist_v)

    pltpu.sync_copy(dst_hbm.at[pl.ds(wid * (E // NW), E // NW)],
                    dst_v.at[pl.ds(0, E // NW)])

    ones = jnp.ones((16,), jnp.float32)

    def acc_body(i, _):
        idx = dst_v[pl.ds(i * 16, 16)]
        plsc.addupdate_scatter(hist_v, [idx], ones)
        return 0

    lax.fori_loop(0, (E // NW) // 16, acc_body, 0)

    pltpu.sync_copy(hist_v, out_hbm.at[pl.ds(wid * HIST, HIST)])


@functools.partial(
    pl.kernel,
    out_type=jax.ShapeDtypeStruct((NC, N_PAD, F), jnp.float32),
    mesh=_mesh,
    scratch_types=[
        pltpu.VMEM((BE,), jnp.int32),
        pltpu.VMEM((BE,), jnp.int32),
        pltpu.VMEM((BE, F), jnp.float32),
        pltpu.VMEM_SHARED((N_PAD, F), jnp.float32),
        pltpu.SemaphoreType.DMA,
    ],
    compiler_params=pltpu.CompilerParams(needs_layout_passes=False),
)
def _agg_kernel(xs_hbm, src_hbm, dst_hbm, out_hbm, sidx, didx, rows, acc, sem):
    c = lax.axis_index("c")
    s = lax.axis_index("s")
    wid = c * NS + s

    pltpu.sync_copy(xs_hbm.at[pl.ds(s * INIT_RPT, INIT_RPT)],
                    acc.at[pl.ds(s * INIT_RPT, INIT_RPT)])
    plsc.subcore_barrier()

    def body(j, _):
        base = wid * EPT + j * BE
        pltpu.sync_copy(src_hbm.at[pl.ds(base, BE)], sidx)
        pltpu.async_copy(xs_hbm.at[sidx], rows, sem).wait()
        pltpu.sync_copy(dst_hbm.at[pl.ds(base, BE)], didx)
        pltpu.sync_copy(rows, acc.at[didx], add=True)
        return 0

    lax.fori_loop(0, NBATCH, body, 0)
    plsc.subcore_barrier()

    pltpu.sync_copy(acc.at[pl.ds(s * OUT_RPT, OUT_RPT)],
                    out_hbm.at[c, pl.ds(s * OUT_RPT, OUT_RPT)])


ROWS_BLK = 1000
GRID = N // ROWS_BLK

_row_spec = pl.BlockSpec((ROWS_BLK, F), lambda i: (i, 0))
_col_spec = pl.BlockSpec((ROWS_BLK, 1), lambda i: (i, 0))
_full_spec = pl.BlockSpec((F, F), lambda i: (0, 0))
_bias_spec = pl.BlockSpec((1, F), lambda i: (0, 0))


def _mm_body(x_ref, w_ref, o_ref):
    o_ref[...] = jnp.dot(x_ref[...], w_ref[...],
                         preferred_element_type=jnp.float32)


def _tc_mm(x, w):
    return pl.pallas_call(
        _mm_body,
        grid=(GRID,),
        in_specs=[_row_spec, _full_spec],
        out_specs=_row_spec,
        out_shape=jax.ShapeDtypeStruct((N, F), jnp.float32),
    )(x, w)


def _degsum_body(d_ref, o_ref):
    o_ref[...] = jnp.sum(d_ref[...], axis=0, keepdims=True)


def _tc_degsum(d):
    return pl.pallas_call(
        _degsum_body,
        grid=(1,),
        in_specs=[pl.BlockSpec((NW, HIST), lambda i: (0, 0))],
        out_specs=pl.BlockSpec((1, HIST), lambda i: (0, 0)),
        out_shape=jax.ShapeDtypeStruct((1, HIST), jnp.float32),
    )(d)


def _prep_body(y_ref, d_ref, xs_ref, dinv_ref):
    deg = d_ref[...] + 1.0
    dinv = lax.rsqrt(deg)
    dinv_ref[...] = dinv
    xs_ref[...] = y_ref[...] * dinv


def _tc_prep(y, d):
    return pl.pallas_call(
        _prep_body,
        grid=(GRID,),
        in_specs=[_row_spec, _col_spec],
        out_specs=[_row_spec, _col_spec],
        out_shape=[jax.ShapeDtypeStruct((N, F), jnp.float32),
                   jax.ShapeDtypeStruct((N, 1), jnp.float32)],
    )(y, d)


def _mid_body(p0_ref, p1_ref, xs_ref, dinv_ref, b_ref, w_ref, o_ref):
    agg = p0_ref[...] + p1_ref[...] - xs_ref[...]
    h = jnp.maximum(agg * dinv_ref[...] + b_ref[...], 0.0)
    o_ref[...] = jnp.dot(h, w_ref[...],
                         preferred_element_type=jnp.float32) * dinv_ref[...]


def _tc_mid(p0, p1, xs, dinv, b, w):
    return pl.pallas_call(
        _mid_body,
        grid=(GRID,),
        in_specs=[_row_spec, _row_spec, _row_spec, _col_spec, _bias_spec,
                  _full_spec],
        out_specs=_row_spec,
        out_shape=jax.ShapeDtypeStruct((N, F), jnp.float32),
    )(p0, p1, xs, dinv, b, w)


def _out_body(q0_ref, q1_ref, xs_ref, dinv_ref, b_ref, o_ref):
    agg = q0_ref[...] + q1_ref[...] - xs_ref[...]
    o_ref[...] = agg * dinv_ref[...] + b_ref[...]


def _tc_out(q0, q1, xs, dinv, b):
    return pl.pallas_call(
        _out_body,
        grid=(GRID,),
        in_specs=[_row_spec, _row_spec, _row_spec, _col_spec, _bias_spec],
        out_specs=_row_spec,
        out_shape=jax.ShapeDtypeStruct((N, F), jnp.float32),
    )(q0, q1, xs, dinv, b)


def kernel(x, edge_index, W1, b1, W2, b2):
    src = edge_index[0]
    dst = edge_index[1]

    pad = jnp.full((E_PAD - E,), N, dtype=jnp.int32)
    src_p = jnp.concatenate([src, pad])
    dst_p = jnp.concatenate([dst, pad])

    d_parts = _deg_kernel(dst, jnp.zeros((HIST,), jnp.float32))
    d = _tc_degsum(d_parts.reshape(NW, HIST))[0, :N].reshape(N, 1)

    zpad = jnp.zeros((N_PAD - N, F), jnp.float32)

    y1 = _tc_mm(x, W1)
    xs1, dinv = _tc_prep(y1, d)
    p = _agg_kernel(jnp.concatenate([xs1, zpad]), src_p, dst_p)
    xs2 = _tc_mid(p[0], p[1], xs1, dinv, b1.reshape(1, F), W2)
    q = _agg_kernel(jnp.concatenate([xs2, zpad]), src_p, dst_p)
    return _tc_out(q[0], q[1], xs2, dinv, b2.reshape(1, F))


# speedup vs baseline: 9.7373x; 1.2040x over previous
import functools

import jax
import jax.numpy as jnp
from jax import lax
from jax.experimental import pallas as pl
from jax.experimental.pallas import tpu as pltpu
from jax.experimental.pallas import tpu_sc as plsc

N = 10000
F = 128
E = 320000

NC = 2
NS = 16
NW = NC * NS

BE = 128
E_PAD = 327680
EPT = E_PAD // NW
NBATCH = EPT // BE
CH = 16
NCHUNK = NBATCH // CH

N_PAD = 10240
INIT_RPT = N_PAD // NS
OUT_RPT = N_PAD // NS

HIST = 10240

_mesh = plsc.VectorSubcoreMesh(core_axis_name="c", subcore_axis_name="s")


@functools.partial(
    pl.kernel,
    out_type=jax.ShapeDtypeStruct((NW * HIST,), jnp.float32),
    mesh=_mesh,
    scratch_types=[
        pltpu.VMEM((HIST,), jnp.float32),
        pltpu.VMEM((EPT,), jnp.int32),
    ],
    compiler_params=pltpu.CompilerParams(needs_layout_passes=False),
)
def _deg_kernel(dst_hbm, zeros_hbm, out_hbm, hist_v, dst_v):
    c = lax.axis_index("c")
    s = lax.axis_index("s")
    wid = c * NS + s

    pltpu.sync_copy(zeros_hbm, hist_v)

    pltpu.sync_copy(dst_hbm.at[pl.ds(wid * (E // NW), E // NW)],
                    dst_v.at[pl.ds(0, E // NW)])

    ones = jnp.ones((16,), jnp.float32)

    def acc_body(i, _):
        idx = dst_v[pl.ds(i * 16, 16)]
        plsc.addupdate_scatter(hist_v, [idx], ones)
        return 0

    lax.fori_loop(0, (E // NW) // 16, acc_body, 0)

    pltpu.sync_copy(hist_v, out_hbm.at[pl.ds(wid * HIST, HIST)])


@functools.partial(
    pl.kernel,
    out_type=jax.ShapeDtypeStruct((NC, N_PAD, F), jnp.float32),
    mesh=_mesh,
    scratch_types=[
        pltpu.VMEM((CH, BE), jnp.int32),
        pltpu.VMEM((CH, BE), jnp.int32),
        pltpu.VMEM((BE, F), jnp.float32),
        pltpu.VMEM((BE, F), jnp.float32),
        pltpu.VMEM_SHARED((N_PAD, F), jnp.float32),
        pltpu.SemaphoreType.DMA,
        pltpu.SemaphoreType.DMA,
    ],
    compiler_params=pltpu.CompilerParams(needs_layout_passes=False),
)
def _agg_kernel(xs_hbm, src_hbm, dst_hbm, out_hbm, sidx, didx, rows_a, rows_b,
                acc, sem_a, sem_b):
    c = lax.axis_index("c")
    s = lax.axis_index("s")
    wid = c * NS + s

    pltpu.sync_copy(xs_hbm.at[pl.ds(s * INIT_RPT, INIT_RPT)],
                    acc.at[pl.ds(s * INIT_RPT, INIT_RPT)])
    plsc.subcore_barrier()

    def chunk_body(ch, _):
        pltpu.sync_copy(src_hbm.at[wid, pl.ds(ch * CH, CH)], sidx)
        pltpu.sync_copy(dst_hbm.at[wid, pl.ds(ch * CH, CH)], didx)
        pltpu.async_copy(xs_hbm.at[sidx.at[0]], rows_a, sem_a)

        def pair(k, _):
            j = 2 * k
            pltpu.async_copy(xs_hbm.at[sidx.at[j + 1]], rows_b, sem_b)
            pltpu.make_async_copy(xs_hbm.at[sidx.at[j]], rows_a, sem_a).wait()
            pltpu.sync_copy(rows_a, acc.at[didx.at[j]], add=True)

            @pl.when(j + 2 < CH)
            def _():
                pltpu.async_copy(xs_hbm.at[sidx.at[j + 2]], rows_a, sem_a)

            pltpu.make_async_copy(xs_hbm.at[sidx.at[j + 1]], rows_b, sem_b).wait()
            pltpu.sync_copy(rows_b, acc.at[didx.at[j + 1]], add=True)
            return 0

        lax.fori_loop(0, CH // 2, pair, 0)
        return 0

    lax.fori_loop(0, NCHUNK, chunk_body, 0)
    plsc.subcore_barrier()

    pltpu.sync_copy(acc.at[pl.ds(s * OUT_RPT, OUT_RPT)],
                    out_hbm.at[c, pl.ds(s * OUT_RPT, OUT_RPT)])


ROWS_BLK = 1000
GRID = N // ROWS_BLK

_row_spec = pl.BlockSpec((ROWS_BLK, F), lambda i: (i, 0))
_col_spec = pl.BlockSpec((ROWS_BLK, 1), lambda i: (i, 0))
_full_spec = pl.BlockSpec((F, F), lambda i: (0, 0))
_bias_spec = pl.BlockSpec((1, F), lambda i: (0, 0))


def _mm_body(x_ref, w_ref, o_ref):
    o_ref[...] = jnp.dot(x_ref[...], w_ref[...],
                         preferred_element_type=jnp.float32)


def _tc_mm(x, w):
    return pl.pallas_call(
        _mm_body,
        grid=(GRID,),
        in_specs=[_row_spec, _full_spec],
        out_specs=_row_spec,
        out_shape=jax.ShapeDtypeStruct((N, F), jnp.float32),
    )(x, w)


def _degsum_body(d_ref, o_ref):
    o_ref[...] = jnp.sum(d_ref[...], axis=0, keepdims=True)


def _tc_degsum(d):
    return pl.pallas_call(
        _degsum_body,
        grid=(1,),
        in_specs=[pl.BlockSpec((NW, HIST), lambda i: (0, 0))],
        out_specs=pl.BlockSpec((1, HIST), lambda i: (0, 0)),
        out_shape=jax.ShapeDtypeStruct((1, HIST), jnp.float32),
    )(d)


def _prep_body(y_ref, d_ref, xs_ref, dinv_ref):
    deg = d_ref[...] + 1.0
    dinv = lax.rsqrt(deg)
    dinv_ref[...] = dinv
    xs_ref[...] = y_ref[...] * dinv


def _tc_prep(y, d):
    return pl.pallas_call(
        _prep_body,
        grid=(GRID,),
        in_specs=[_row_spec, _col_spec],
        out_specs=[_row_spec, _col_spec],
        out_shape=[jax.ShapeDtypeStruct((N, F), jnp.float32),
                   jax.ShapeDtypeStruct((N, 1), jnp.float32)],
    )(y, d)


def _mid_body(p0_ref, p1_ref, xs_ref, dinv_ref, b_ref, w_ref, o_ref):
    agg = p0_ref[...] + p1_ref[...] - xs_ref[...]
    h = jnp.maximum(agg * dinv_ref[...] + b_ref[...], 0.0)
    o_ref[...] = jnp.dot(h, w_ref[...],
                         preferred_element_type=jnp.float32) * dinv_ref[...]


def _tc_mid(p0, p1, xs, dinv, b, w):
    return pl.pallas_call(
        _mid_body,
        grid=(GRID,),
        in_specs=[_row_spec, _row_spec, _row_spec, _col_spec, _bias_spec,
                  _full_spec],
        out_specs=_row_spec,
        out_shape=jax.ShapeDtypeStruct((N, F), jnp.float32),
    )(p0, p1, xs, dinv, b, w)


def _out_body(q0_ref, q1_ref, xs_ref, dinv_ref, b_ref, o_ref):
    agg = q0_ref[...] + q1_ref[...] - xs_ref[...]
    o_ref[...] = agg * dinv_ref[...] + b_ref[...]


def _tc_out(q0, q1, xs, dinv, b):
    return pl.pallas_call(
        _out_body,
        grid=(GRID,),
        in_specs=[_row_spec, _row_spec, _row_spec, _col_spec, _bias_spec],
        out_specs=_row_spec,
        out_shape=jax.ShapeDtypeStruct((N, F), jnp.float32),
    )(q0, q1, xs, dinv, b)


def kernel(x, edge_index, W1, b1, W2, b2):
    src = edge_index[0]
    dst = edge_index[1]

    pad = jnp.full((E_PAD - E,), N, dtype=jnp.int32)
    src_p = jnp.concatenate([src, pad]).reshape(NW, NBATCH, BE)
    dst_p = jnp.concatenate([dst, pad]).reshape(NW, NBATCH, BE)

    d_parts = _deg_kernel(dst, jnp.zeros((HIST,), jnp.float32))
    d = _tc_degsum(d_parts.reshape(NW, HIST))[0, :N].reshape(N, 1)

    zpad = jnp.zeros((N_PAD - N, F), jnp.float32)

    y1 = _tc_mm(x, W1)
    xs1, dinv = _tc_prep(y1, d)
    p = _agg_kernel(jnp.concatenate([xs1, zpad]), src_p, dst_p)
    xs2 = _tc_mid(p[0], p[1], xs1, dinv, b1.reshape(1, F), W2)
    q = _agg_kernel(jnp.concatenate([xs2, zpad]), src_p, dst_p)
    return _tc_out(q[0], q[1], xs2, dinv, b2.reshape(1, F))


# speedup vs baseline: 10.1192x; 1.0392x over previous
import functools

import jax
import jax.numpy as jnp
from jax import lax
from jax.experimental import pallas as pl
from jax.experimental.pallas import tpu as pltpu
from jax.experimental.pallas import tpu_sc as plsc

N = 10000
F = 128
E = 320000

NC = 2
NS = 16
NW = NC * NS

BE = 128
E_PAD = 327680
NB_TOT = E_PAD // BE
NB_C0 = 128
NB_C1 = 32
CH = 16

N_PAD = 10240
INIT_RPT = N_PAD // NS
OUT_RPT = N_PAD // NS

HIST = 10240

_mesh = plsc.VectorSubcoreMesh(core_axis_name="c", subcore_axis_name="s")


@functools.partial(
    pl.kernel,
    out_type=jax.ShapeDtypeStruct((NW * HIST,), jnp.float32),
    mesh=_mesh,
    scratch_types=[
        pltpu.VMEM((HIST,), jnp.float32),
        pltpu.VMEM((E // NW,), jnp.int32),
    ],
    compiler_params=pltpu.CompilerParams(needs_layout_passes=False),
)
def _deg_kernel(dst_hbm, zeros_hbm, out_hbm, hist_v, dst_v):
    c = lax.axis_index("c")
    s = lax.axis_index("s")
    wid = c * NS + s

    pltpu.sync_copy(zeros_hbm, hist_v)

    pltpu.sync_copy(dst_hbm.at[pl.ds(wid * (E // NW), E // NW)],
                    dst_v.at[pl.ds(0, E // NW)])

    ones = jnp.ones((16,), jnp.float32)

    def acc_body(i, _):
        idx = dst_v[pl.ds(i * 16, 16)]
        plsc.addupdate_scatter(hist_v, [idx], ones)
        return 0

    lax.fori_loop(0, (E // NW) // 16, acc_body, 0)

    pltpu.sync_copy(hist_v, out_hbm.at[pl.ds(wid * HIST, HIST)])


@functools.partial(
    pl.kernel,
    out_type=jax.ShapeDtypeStruct((NC, N_PAD, F), jnp.float32),
    mesh=_mesh,
    scratch_types=[
        pltpu.VMEM((CH, BE), jnp.int32),
        pltpu.VMEM((CH, BE), jnp.int32),
        pltpu.VMEM((BE, F), jnp.float32),
        pltpu.VMEM((BE, F), jnp.float32),
        pltpu.VMEM_SHARED((N_PAD, F), jnp.float32),
        pltpu.SemaphoreType.DMA,
        pltpu.SemaphoreType.DMA,
    ],
    compiler_params=pltpu.CompilerParams(needs_layout_passes=False),
)
def _agg_kernel(xs_hbm, src_hbm, dst_hbm, out_hbm, sidx, didx, rows_a, rows_b,
                acc, sem_a, sem_b):
    c = lax.axis_index("c")
    s = lax.axis_index("s")
    wid = c * NS + s

    pltpu.sync_copy(xs_hbm.at[pl.ds(s * INIT_RPT, INIT_RPT)],
                    acc.at[pl.ds(s * INIT_RPT, INIT_RPT)])
    plsc.subcore_barrier()

    row0 = jnp.where(c == 0, s * NB_C0, NS * NB_C0 + s * NB_C1)
    nchunk = jnp.where(c == 0, NB_C0 // CH, NB_C1 // CH)

    def chunk_body(ch, _):
        pltpu.sync_copy(src_hbm.at[pl.ds(row0 + ch * CH, CH)], sidx)
        pltpu.sync_copy(dst_hbm.at[pl.ds(row0 + ch * CH, CH)], didx)
        pltpu.async_copy(xs_hbm.at[sidx.at[0]], rows_a, sem_a)

        def pair(k, _):
            j = 2 * k
            pltpu.async_copy(xs_hbm.at[sidx.at[j + 1]], rows_b, sem_b)
            pltpu.make_async_copy(xs_hbm.at[sidx.at[j]], rows_a, sem_a).wait()
            pltpu.sync_copy(rows_a, acc.at[didx.at[j]], add=True)

            @pl.when(j + 2 < CH)
            def _():
                pltpu.async_copy(xs_hbm.at[sidx.at[j + 2]], rows_a, sem_a)

            pltpu.make_async_copy(xs_hbm.at[sidx.at[j + 1]], rows_b, sem_b).wait()
            pltpu.sync_copy(rows_b, acc.at[didx.at[j + 1]], add=True)
            return 0

        lax.fori_loop(0, CH // 2, pair, 0)
        return 0

    lax.fori_loop(0, nchunk, chunk_body, 0)
    plsc.subcore_barrier()

    pltpu.sync_copy(acc.at[pl.ds(s * OUT_RPT, OUT_RPT)],
                    out_hbm.at[c, pl.ds(s * OUT_RPT, OUT_RPT)])


ROWS_BLK = 1000
GRID = N // ROWS_BLK

_row_spec = pl.BlockSpec((ROWS_BLK, F), lambda i: (i, 0))
_col_spec = pl.BlockSpec((ROWS_BLK, 1), lambda i: (i, 0))
_full_spec = pl.BlockSpec((F, F), lambda i: (0, 0))
_bias_spec = pl.BlockSpec((1, F), lambda i: (0, 0))


def _mm_body(x_ref, w_ref, o_ref):
    o_ref[...] = jnp.dot(x_ref[...], w_ref[...],
                         preferred_element_type=jnp.float32)


def _tc_mm(x, w):
    return pl.pallas_call(
        _mm_body,
        grid=(GRID,),
        in_specs=[_row_spec, _full_spec],
        out_specs=_row_spec,
        out_shape=jax.ShapeDtypeStruct((N, F), jnp.float32),
    )(x, w)


def _degsum_body(d_ref, o_ref):
    o_ref[...] = jnp.sum(d_ref[...], axis=0, keepdims=True)


def _tc_degsum(d):
    return pl.pallas_call(
        _degsum_body,
        grid=(1,),
        in_specs=[pl.BlockSpec((NW, HIST), lambda i: (0, 0))],
        out_specs=pl.BlockSpec((1, HIST), lambda i: (0, 0)),
        out_shape=jax.ShapeDtypeStruct((1, HIST), jnp.float32),
    )(d)


def _prep_body(y_ref, d_ref, xs_ref, dinv_ref):
    deg = d_ref[...] + 1.0
    dinv = lax.rsqrt(deg)
    dinv_ref[...] = dinv
    xs_ref[...] = y_ref[...] * dinv


def _tc_prep(y, d):
    return pl.pallas_call(
        _prep_body,
        grid=(GRID,),
        in_specs=[_row_spec, _col_spec],
        out_specs=[_row_spec, _col_spec],
        out_shape=[jax.ShapeDtypeStruct((N, F), jnp.float32),
                   jax.ShapeDtypeStruct((N, 1), jnp.float32)],
    )(y, d)


def _mid_body(p0_ref, p1_ref, xs_ref, dinv_ref, b_ref, w_ref, o_ref):
    agg = p0_ref[...] + p1_ref[...] - xs_ref[...]
    h = jnp.maximum(agg * dinv_ref[...] + b_ref[...], 0.0)
    o_ref[...] = jnp.dot(h, w_ref[...],
                         preferred_element_type=jnp.float32) * dinv_ref[...]


def _tc_mid(p0, p1, xs, dinv, b, w):
    return pl.pallas_call(
        _mid_body,
        grid=(GRID,),
        in_specs=[_row_spec, _row_spec, _row_spec, _col_spec, _bias_spec,
                  _full_spec],
        out_specs=_row_spec,
        out_shape=jax.ShapeDtypeStruct((N, F), jnp.float32),
    )(p0, p1, xs, dinv, b, w)


def _out_body(q0_ref, q1_ref, xs_ref, dinv_ref, b_ref, o_ref):
    agg = q0_ref[...] + q1_ref[...] - xs_ref[...]
    o_ref[...] = agg * dinv_ref[...] + b_ref[...]


def _tc_out(q0, q1, xs, dinv, b):
    return pl.pallas_call(
        _out_body,
        grid=(GRID,),
        in_specs=[_row_spec, _row_spec, _row_spec, _col_spec, _bias_spec],
        out_specs=_row_spec,
        out_shape=jax.ShapeDtypeStruct((N, F), jnp.float32),
    )(q0, q1, xs, dinv, b)


def kernel(x, edge_index, W1, b1, W2, b2):
    src = edge_index[0]
    dst = edge_index[1]

    pad = jnp.full((E_PAD - E,), N, dtype=jnp.int32)
    src_p = jnp.concatenate([src, pad]).reshape(NB_TOT, BE)
    dst_p = jnp.concatenate([dst, pad]).reshape(NB_TOT, BE)

    d_parts = _deg_kernel(dst, jnp.zeros((HIST,), jnp.float32))
    d = _tc_degsum(d_parts.reshape(NW, HIST))[0, :N].reshape(N, 1)

    zpad = jnp.zeros((N_PAD - N, F), jnp.float32)

    y1 = _tc_mm(x, W1)
    xs1, dinv = _tc_prep(y1, d)
    p = _agg_kernel(jnp.concatenate([xs1, zpad]), src_p, dst_p)
    xs2 = _tc_mid(p[0], p[1], xs1, dinv, b1.reshape(1, F), W2)
    q = _agg_kernel(jnp.concatenate([xs2, zpad]), src_p, dst_p)
    return _tc_out(q[0], q[1], xs2, dinv, b2.reshape(1, F))


# speedup vs baseline: 11.9704x; 1.1829x over previous
import functools

import jax
import jax.numpy as jnp
from jax import lax
from jax.experimental import pallas as pl
from jax.experimental.pallas import tpu as pltpu
from jax.experimental.pallas import tpu_sc as plsc

N = 10000
F = 128
E = 320000

NC = 2
NS = 16
NW = NC * NS

BE = 128
E_PAD = 327680
NB_TOT = E_PAD // BE
NB_C0 = 152
NB_C1 = 8
CH = 8

N_PAD = 10240
INIT_RPT = N_PAD // NS
OUT_RPT = N_PAD // NS

HIST = 10240

_mesh = plsc.VectorSubcoreMesh(core_axis_name="c", subcore_axis_name="s")


@functools.partial(
    pl.kernel,
    out_type=jax.ShapeDtypeStruct((NW * HIST,), jnp.float32),
    mesh=_mesh,
    scratch_types=[
        pltpu.VMEM((HIST,), jnp.float32),
        pltpu.VMEM((E // NW,), jnp.int32),
    ],
    compiler_params=pltpu.CompilerParams(needs_layout_passes=False),
)
def _deg_kernel(dst_hbm, zeros_hbm, out_hbm, hist_v, dst_v):
    c = lax.axis_index("c")
    s = lax.axis_index("s")
    wid = c * NS + s

    pltpu.sync_copy(zeros_hbm, hist_v)

    pltpu.sync_copy(dst_hbm.at[pl.ds(wid * (E // NW), E // NW)],
                    dst_v.at[pl.ds(0, E // NW)])

    ones = jnp.ones((16,), jnp.float32)

    def acc_body(i, _):
        idx = dst_v[pl.ds(i * 16, 16)]
        plsc.addupdate_scatter(hist_v, [idx], ones)
        return 0

    lax.fori_loop(0, (E // NW) // 16, acc_body, 0)

    pltpu.sync_copy(hist_v, out_hbm.at[pl.ds(wid * HIST, HIST)])


@functools.partial(
    pl.kernel,
    out_type=jax.ShapeDtypeStruct((NC, N_PAD, F), jnp.float32),
    mesh=_mesh,
    scratch_types=[
        pltpu.VMEM((CH, BE), jnp.int32),
        pltpu.VMEM((CH, BE), jnp.int32),
        pltpu.VMEM((BE, F), jnp.float32),
        pltpu.VMEM((BE, F), jnp.float32),
        pltpu.VMEM_SHARED((N_PAD, F), jnp.float32),
        pltpu.SemaphoreType.DMA,
        pltpu.SemaphoreType.DMA,
    ],
    compiler_params=pltpu.CompilerParams(needs_layout_passes=False),
)
def _agg_kernel(xs_hbm, src_hbm, dst_hbm, out_hbm, sidx, didx, rows_a, rows_b,
                acc, sem_a, sem_b):
    c = lax.axis_index("c")
    s = lax.axis_index("s")

    pltpu.sync_copy(xs_hbm.at[pl.ds(s * INIT_RPT, INIT_RPT)],
                    acc.at[pl.ds(s * INIT_RPT, INIT_RPT)])
    plsc.subcore_barrier()

    row0 = jnp.where(c == 0, s * NB_C0, NS * NB_C0 + s * NB_C1)
    nchunk = jnp.where(c == 0, NB_C0 // CH, NB_C1 // CH)

    def chunk_body(ch, _):
        pltpu.sync_copy(src_hbm.at[pl.ds(row0 + ch * CH, CH)], sidx)
        pltpu.sync_copy(dst_hbm.at[pl.ds(row0 + ch * CH, CH)], didx)
        pltpu.async_copy(xs_hbm.at[sidx.at[0]], rows_a, sem_a)

        def pair(k, _):
            j = 2 * k
            pltpu.async_copy(xs_hbm.at[sidx.at[j + 1]], rows_b, sem_b)
            pltpu.make_async_copy(xs_hbm.at[sidx.at[j]], rows_a, sem_a).wait()
            pltpu.sync_copy(rows_a, acc.at[didx.at[j]], add=True)

            @pl.when(j + 2 < CH)
            def _():
                pltpu.async_copy(xs_hbm.at[sidx.at[j + 2]], rows_a, sem_a)

            pltpu.make_async_copy(xs_hbm.at[sidx.at[j + 1]], rows_b, sem_b).wait()
            pltpu.sync_copy(rows_b, acc.at[didx.at[j + 1]], add=True)
            return 0

        lax.fori_loop(0, CH // 2, pair, 0)
        return 0

    lax.fori_loop(0, nchunk, chunk_body, 0)
    plsc.subcore_barrier()

    pltpu.sync_copy(acc.at[pl.ds(s * OUT_RPT, OUT_RPT)],
                    out_hbm.at[c, pl.ds(s * OUT_RPT, OUT_RPT)])


ROWS_BLK = 1000
GRID = N // ROWS_BLK

_row_spec = pl.BlockSpec((ROWS_BLK, F), lambda i: (i, 0))
_col_spec = pl.BlockSpec((ROWS_BLK, 1), lambda i: (i, 0))
_full_spec = pl.BlockSpec((F, F), lambda i: (0, 0))
_bias_spec = pl.BlockSpec((1, F), lambda i: (0, 0))


def _mm_body(x_ref, w_ref, o_ref):
    o_ref[...] = jnp.dot(x_ref[...], w_ref[...],
                         preferred_element_type=jnp.float32)


def _tc_mm(x, w):
    return pl.pallas_call(
        _mm_body,
        grid=(GRID,),
        in_specs=[_row_spec, _full_spec],
        out_specs=_row_spec,
        out_shape=jax.ShapeDtypeStruct((N, F), jnp.float32),
    )(x, w)


def _degsum_body(d_ref, o_ref):
    o_ref[...] = jnp.sum(d_ref[...], axis=0, keepdims=True)


def _tc_degsum(d):
    return pl.pallas_call(
        _degsum_body,
        grid=(1,),
        in_specs=[pl.BlockSpec((NW, HIST), lambda i: (0, 0))],
        out_specs=pl.BlockSpec((1, HIST), lambda i: (0, 0)),
        out_shape=jax.ShapeDtypeStruct((1, HIST), jnp.float32),
    )(d)


def _prep_body(y_ref, d_ref, xs_ref, dinv_ref):
    deg = d_ref[...] + 1.0
    dinv = lax.rsqrt(deg)
    dinv_ref[...] = dinv
    xs_ref[...] = y_ref[...] * dinv


def _tc_prep(y, d):
    return pl.pallas_call(
        _prep_body,
        grid=(GRID,),
        in_specs=[_row_spec, _col_spec],
        out_specs=[_row_spec, _col_spec],
        out_shape=[jax.ShapeDtypeStruct((N, F), jnp.float32),
                   jax.ShapeDtypeStruct((N, 1), jnp.float32)],
    )(y, d)


def _mid_body(p0_ref, p1_ref, xs_ref, dinv_ref, b_ref, w_ref, o_ref):
    agg = p0_ref[...] + p1_ref[...] - xs_ref[...]
    h = jnp.maximum(agg * dinv_ref[...] + b_ref[...], 0.0)
    o_ref[...] = jnp.dot(h, w_ref[...],
                         preferred_element_type=jnp.float32) * dinv_ref[...]


def _tc_mid(p0, p1, xs, dinv, b, w):
    return pl.pallas_call(
        _mid_body,
        grid=(GRID,),
        in_specs=[_row_spec, _row_spec, _row_spec, _col_spec, _bias_spec,
                  _full_spec],
        out_specs=_row_spec,
        out_shape=jax.ShapeDtypeStruct((N, F), jnp.float32),
    )(p0, p1, xs, dinv, b, w)


def _out_body(q0_ref, q1_ref, xs_ref, dinv_ref, b_ref, o_ref):
    agg = q0_ref[...] + q1_ref[...] - xs_ref[...]
    o_ref[...] = agg * dinv_ref[...] + b_ref[...]


def _tc_out(q0, q1, xs, dinv, b):
    return pl.pallas_call(
        _out_body,
        grid=(GRID,),
        in_specs=[_row_spec, _row_spec, _row_spec, _col_spec, _bias_spec],
        out_specs=_row_spec,
        out_shape=jax.ShapeDtypeStruct((N, F), jnp.float32),
    )(q0, q1, xs, dinv, b)


def kernel(x, edge_index, W1, b1, W2, b2):
    src = edge_index[0]
    dst = edge_index[1]

    pad = jnp.full((E_PAD - E,), N, dtype=jnp.int32)
    src_p = jnp.concatenate([src, pad]).reshape(NB_TOT, BE)
    dst_p = jnp.concatenate([dst, pad]).reshape(NB_TOT, BE)

    d_parts = _deg_kernel(dst, jnp.zeros((HIST,), jnp.float32))
    d = _tc_degsum(d_parts.reshape(NW, HIST))[0, :N].reshape(N, 1)

    zpad = jnp.zeros((N_PAD - N, F), jnp.float32)

    y1 = _tc_mm(x, W1)
    xs1, dinv = _tc_prep(y1, d)
    p = _agg_kernel(jnp.concatenate([xs1, zpad]), src_p, dst_p)
    xs2 = _tc_mid(p[0, :N], p[1, :N], xs1, dinv, b1.reshape(1, F), W2)
    q = _agg_kernel(jnp.concatenate([xs2, zpad]), src_p, dst_p)
    return _tc_out(q[0, :N], q[1, :N], xs2, dinv, b2.reshape(1, F))
